# baseline (speedup 1.0000x reference)
import functools

import jax
import jax.numpy as jnp
from jax import lax
from jax.experimental import pallas as pl
from jax.experimental.pallas import tpu as pltpu
from jax.experimental.pallas import tpu_sc as plsc

_NUM_OBJ = 151
_NUM_RELS = 51
_BATCH = 16384

_INFO = plsc.get_sparse_core_info()
_NC = _INFO.num_cores
_NS = _INFO.num_subcores
_NW = _NC * _NS
_L = _INFO.num_lanes
_ROWS = _BATCH // _NW
_CHUNKS = _ROWS // _L
_GATHER_W = 128
_NGATHER = _ROWS // _GATHER_W
_OUT_W = _ROWS * _NUM_RELS


def _body(labels_hbm, kb_hbm, out_hbm, labels_v, pairid_v, keys_v, out_v, sem):
    wid = lax.axis_index("s") * _NC + lax.axis_index("c")
    iota = lax.iota(jnp.int32, _L)

    lbase = pl.multiple_of(wid * (2 * _ROWS), 2 * _ROWS)
    pltpu.sync_copy(labels_hbm.at[pl.ds(lbase, 2 * _ROWS)], labels_v)

    for c in range(_CHUNKS):
        sidx = (c * _L + iota) * 2
        subj = plsc.load_gather(labels_v, [sidx])
        obj = plsc.load_gather(labels_v, [sidx + 1])
        pairid_v[pl.ds(c * _L, _L)] = subj * _NUM_OBJ + obj

    copies = [
        pltpu.async_copy(
            kb_hbm.at[pairid_v.at[pl.ds(j * _GATHER_W, _GATHER_W)]],
            keys_v.at[pl.ds(j * _GATHER_W, _GATHER_W)],
            sem,
        )
        for j in range(_NGATHER)
    ]

    zeros = jnp.zeros((_L,), jnp.float32)
    unroll = 8
    span = unroll * _L

    def _zero(i, carry):
        b0 = pl.multiple_of(i * span, span)
        for j in range(unroll):
            out_v[pl.ds(b0 + j * _L, _L)] = zeros
        return carry

    lax.fori_loop(0, _OUT_W // span, _zero, 0)

    for cp in copies:
        cp.wait()

    ones = jnp.full((_L,), 1.0, jnp.float32)
    for c in range(_CHUNKS):
        keys = keys_v[pl.ds(c * _L, _L)]
        flat = (c * _L + iota) * _NUM_RELS + keys
        plsc.store_scatter(out_v, [flat], ones)

    obase = pl.multiple_of(wid * _OUT_W, 8)
    pltpu.sync_copy(out_v, out_hbm.at[pl.ds(obase, _OUT_W)])


@jax.jit
def _kb_bias_sc(labels_flat, kb_table):
    mesh = plsc.VectorSubcoreMesh(core_axis_name="c", subcore_axis_name="s")
    run = functools.partial(
        pl.kernel,
        out_type=jax.ShapeDtypeStruct((_BATCH * _NUM_RELS,), jnp.float32),
        mesh=mesh,
        compiler_params=pltpu.CompilerParams(
            needs_layout_passes=False,
            skip_device_barrier=True,
            disable_bounds_checks=True,
            disable_semaphore_checks=True,
        ),
        scratch_types=[
            pltpu.VMEM((2 * _ROWS,), jnp.int32),
            pltpu.VMEM((_ROWS,), jnp.int32),
            pltpu.VMEM((_ROWS,), jnp.int32),
            pltpu.VMEM((_OUT_W,), jnp.float32),
            pltpu.SemaphoreType.DMA,
        ],
    )(_body)
    return run(labels_flat, kb_table)


def kernel(labels, kb_table):
    out_flat = _kb_bias_sc(labels.reshape(-1), kb_table)
    return out_flat.reshape(_BATCH, _NUM_RELS)


# speedup vs baseline: 2.2340x; 2.2340x over previous
import functools

import jax
import jax.numpy as jnp
from jax import lax
from jax.experimental import pallas as pl
from jax.experimental.pallas import tpu as pltpu
from jax.experimental.pallas import tpu_sc as plsc

_NUM_OBJ = 151
_NUM_RELS = 51
_BATCH = 16384

_INFO = plsc.get_sparse_core_info()
_NC = _INFO.num_cores
_NS = _INFO.num_subcores
_NW = _NC * _NS
_L = _INFO.num_lanes
_COLS = _BATCH // _NW
_CHUNKS = _COLS // _L
_GATHER_W = 128
_NGATHER = _COLS // _GATHER_W


def _body(labels_hbm, kb_hbm, out_hbm, labels_v, pairid_v, keys_v, out_v, sem):
    wid = lax.axis_index("s") * _NC + lax.axis_index("c")
    iota = lax.iota(jnp.int32, _L)
    cbase = pl.multiple_of(wid * _COLS, _COLS)

    pltpu.sync_copy(labels_hbm.at[:, pl.ds(cbase, _COLS)], labels_v)

    for c in range(_CHUNKS):
        subj = labels_v[0, pl.ds(c * _L, _L)]
        obj = labels_v[1, pl.ds(c * _L, _L)]
        pairid_v[pl.ds(c * _L, _L)] = subj * _NUM_OBJ + obj

    copies = [
        pltpu.async_copy(
            kb_hbm.at[pairid_v.at[pl.ds(j * _GATHER_W, _GATHER_W)]],
            keys_v.at[pl.ds(j * _GATHER_W, _GATHER_W)],
            sem,
        )
        for j in range(_NGATHER)
    ]

    zeros = jnp.zeros((_L,), jnp.float32)

    def _zero(j, carry):
        for b in range(_COLS // _L):
            out_v[j, pl.ds(b * _L, _L)] = zeros
        return carry

    lax.fori_loop(0, _NUM_RELS, _zero, 0)

    for cp in copies:
        cp.wait()

    ones = jnp.full((_L,), 1.0, jnp.float32)
    for c in range(_CHUNKS):
        keys = keys_v[pl.ds(c * _L, _L)]
        plsc.store_scatter(out_v, [keys, c * _L + iota], ones)

    for j0 in range(0, _NUM_RELS, 8):
        h = min(8, _NUM_RELS - j0)
        pltpu.sync_copy(
            out_v.at[pl.ds(j0, h), :],
            out_hbm.at[pl.ds(j0, h), pl.ds(cbase, _COLS)],
        )


@jax.jit
def _kb_bias_sc(labels_t, kb_table):
    mesh = plsc.VectorSubcoreMesh(core_axis_name="c", subcore_axis_name="s")
    run = functools.partial(
        pl.kernel,
        out_type=jax.ShapeDtypeStruct((_NUM_RELS, _BATCH), jnp.float32),
        mesh=mesh,
        compiler_params=pltpu.CompilerParams(
            needs_layout_passes=False,
            skip_device_barrier=True,
            disable_bounds_checks=True,
            disable_semaphore_checks=True,
        ),
        scratch_types=[
            pltpu.VMEM((2, _COLS), jnp.int32),
            pltpu.VMEM((_COLS,), jnp.int32),
            pltpu.VMEM((_COLS,), jnp.int32),
            pltpu.VMEM((_NUM_RELS, _COLS), jnp.float32),
            pltpu.SemaphoreType.DMA,
        ],
    )(_body)
    return run(labels_t, kb_table)


def kernel(labels, kb_table):
    return _kb_bias_sc(labels.T, kb_table).T
